# baseline (speedup 1.0000x reference)
import jax
import jax.numpy as jnp
from jax import lax
from jax.experimental import pallas as pl
from jax.experimental.pallas import tpu as pltpu

MEM_ROWS = 100000
CHUNK = 1000
NBUF = 10
NGROUPS = MEM_ROWS // (CHUNK * NBUF)
KDIM = 128
VDIM = 512


def _soft_read_kernel(x_ref, fz_ref, k_hbm, v_hbm, o_ref, *bufs):
    kbufs = bufs[:NBUF]
    vbufs = bufs[NBUF : 2 * NBUF]
    sems = bufs[2 * NBUF]

    def kcopy(i, b):
        return pltpu.make_async_copy(
            k_hbm.at[pl.ds(i * CHUNK, CHUNK), :], kbufs[b], sems.at[b, 0]
        )

    def vcopy(i, b):
        return pltpu.make_async_copy(
            v_hbm.at[pl.ds(i * CHUNK, CHUNK), :], vbufs[b], sems.at[b, 1]
        )

    for b in range(NBUF):
        kcopy(b, b).start()
        vcopy(b, b).start()

    x = x_ref[...]
    x_norm = jnp.sqrt(jnp.sum(x * x))
    ones = jnp.ones((1, KDIM), jnp.float32)

    def group(g, carry):
        acc, den = carry
        for b in range(NBUF):
            i = g * NBUF + b
            kcopy(i, b).wait()
            vcopy(i, b).wait()
            k = kbufs[b][...]
            v = vbufs[b][...]
            kt = k.T
            num = jnp.dot(x, kt)
            sq = jnp.dot(ones, kt * kt)
            denom = jnp.maximum(x_norm * jnp.sqrt(sq), 1e-6)
            p = jnp.exp(num / denom)
            part = jnp.dot(p, v)
            psum = jnp.sum(p)

            @pl.when(i + NBUF < NGROUPS * NBUF)
            def _next():
                kcopy(i + NBUF, b).start()
                vcopy(i + NBUF, b).start()

            acc = acc + part
            den = den + psum
        return (acc, den)

    acc, den = jax.lax.fori_loop(
        0, NGROUPS, group, (jnp.zeros((1, VDIM), jnp.float32), jnp.float32(0.0))
    )
    o_ref[...] = 0.7 * (acc / den) + 0.3 * fz_ref[...]


@jax.jit
def _soft_read(x_key, f_z_value, key_memory, value_memory):
    m, kdim = key_memory.shape
    v2d = value_memory.reshape(m, VDIM)
    fz2d = f_z_value.reshape(1, VDIM)

    out = pl.pallas_call(
        _soft_read_kernel,
        in_specs=[
            pl.BlockSpec(memory_space=pltpu.MemorySpace.VMEM),
            pl.BlockSpec(memory_space=pltpu.MemorySpace.VMEM),
            pl.BlockSpec(memory_space=pltpu.MemorySpace.HBM),
            pl.BlockSpec(memory_space=pltpu.MemorySpace.HBM),
        ],
        out_specs=pl.BlockSpec(memory_space=pltpu.MemorySpace.VMEM),
        out_shape=jax.ShapeDtypeStruct((1, VDIM), jnp.float32),
        scratch_shapes=(
            [pltpu.VMEM((CHUNK, KDIM), jnp.float32) for _ in range(NBUF)]
            + [pltpu.VMEM((CHUNK, VDIM), jnp.float32) for _ in range(NBUF)]
            + [pltpu.SemaphoreType.DMA((NBUF, 2))]
        ),
    )(x_key, fz2d, key_memory, v2d)
    return out.reshape(f_z_value.shape)


def kernel(x_key, f_z_value, key_memory, value_memory):
    return _soft_read(x_key, f_z_value, key_memory, value_memory)
